# baseline (speedup 1.0000x reference)
import functools

import jax
import jax.numpy as jnp
from jax import lax
from jax.experimental import pallas as pl
from jax.experimental.pallas import tpu as pltpu
from jax.experimental.pallas import tpu_sc as plsc

_F = 26
_V = 100000
_D = 32
_B = 16384
_NW = 32
_BPW = _B // _NW
_Q = 128
_G = 128 // _D
_L = 16
_FG = 7
_NT = _FG * (_BPW // _Q)
_IDXR = _F * _BPW // 128


def _sc_encode(idx_all, tab128):
    mesh = plsc.VectorSubcoreMesh(core_axis_name="c", subcore_axis_name="s")

    @functools.partial(
        pl.kernel,
        mesh=mesh,
        out_type=jax.ShapeDtypeStruct((_B, _FG * 128), jnp.float32),
        scratch_types=[
            pltpu.VMEM((_IDXR, 128), jnp.int32),
            pltpu.VMEM((2, _Q), jnp.int32),
            pltpu.VMEM((2, _Q, 128), jnp.float32),
            pltpu.VMEM((2, _Q, 128), jnp.float32),
        ] + [pltpu.SemaphoreType.DMA] * 4,
        compiler_params=pltpu.CompilerParams(use_tc_tiling_on_sc=True, needs_layout_passes=False),
    )
    def k(idx_hbm, tab_hbm, out_hbm, idx_v, mrow_v, gbuf, asm, *sems):
        gsem = sems[:2]
        ssem = sems[2:]
        wid = lax.axis_index("s") * 2 + lax.axis_index("c")
        base = wid * _BPW
        lane = lax.broadcasted_iota(jnp.int32, (_L,), 0)

        pltpu.sync_copy(idx_hbm.at[wid], idx_v)

        def slot_field(t, j):
            fg = t // 4
            return jnp.where(fg < _FG - 1, fg * 4 + j, 24 + (j % 2))

        def idx_chunk(t, j, g):
            row = slot_field(t, j) * 4 + (t % 4)
            return idx_v[row, pl.ds(g * _L, _L)]

        def prep_fire(t, j):
            m = j % 2
            for g in range(_Q // _L):
                v = idx_chunk(t, j, g)
                mrow_v[m, pl.ds(g * _L, _L)] = lax.shift_right_logical(v, 2)
            pltpu.async_copy(tab_hbm.at[mrow_v.at[m]], gbuf.at[m], gsem[m])

        def wait_gather(j):
            m = j % 2
            pltpu.make_async_copy(tab_hbm.at[mrow_v.at[m]], gbuf.at[m],
                                  gsem[m]).wait()

        def compact(t, j, r):
            m = j % 2

            def body(g, carry):
                v = idx_chunk(t, j, g)
                rows = lane + g * _L
                quarter = lax.bitwise_and(v, _G - 1) * _D
                for d in range(_D):
                    vals = plsc.load_gather(gbuf.at[m], [rows, quarter + d])
                    plsc.store_scatter(asm.at[r],
                                       [rows, lane * 0 + (j * _D + d)], vals)
                return carry

            lax.fori_loop(0, _Q // _L, body, 0, unroll=False)

        def out_slice(t):
            fg, q = t // 4, t % 4
            return out_hbm.at[pl.ds(base + q * _Q, _Q),
                              pl.ds(fg * 128, 128)]

        def fire_store(t, r):
            pltpu.async_copy(asm.at[r], out_slice(t), ssem[r])

        def wait_store(r):
            pltpu.make_async_copy(asm.at[r], out_slice(0), ssem[r]).wait()

        prep_fire(0, 0)
        prep_fire(0, 1)

        def task_body(t, r):
            @pl.when(t >= 2)
            def _():
                wait_store(r)

            for j in range(4):
                wait_gather(j)
                compact(t, j, r)
                nj = j + 2
                if nj < 4:
                    prep_fire(t, nj)
                else:

                    @pl.when(t + 1 < _NT)
                    def _():
                        prep_fire(t + 1, nj - 4)

            fire_store(t, r)

        def pair_body(p, carry):
            task_body(2 * p, 0)
            task_body(2 * p + 1, 1)
            return carry

        lax.fori_loop(0, _NT // 2, pair_body, 0, unroll=False)
        wait_store(0)
        wait_store(1)

    return k(idx_all, tab128)


def kernel(sparse_tensors, tables):
    idx = sparse_tensors.astype(jnp.int32)
    offs = (jnp.arange(_F, dtype=jnp.int32) * _V)[:, None]
    gidx = (
        (idx + offs)
        .reshape(_F, _NW, _BPW)
        .transpose(1, 0, 2)
        .reshape(_NW, _IDXR, 128)
    )
    tab128 = tables.reshape(_F * _V // _G, 128)
    out_pad = _sc_encode(gidx, tab128)
    return out_pad[:, : _F * _D]


# speedup vs baseline: 1.3333x; 1.3333x over previous
import functools

import jax
import jax.numpy as jnp
from jax import lax
from jax.experimental import pallas as pl
from jax.experimental.pallas import tpu as pltpu
from jax.experimental.pallas import tpu_sc as plsc

_F = 26
_V = 100000
_D = 32
_B = 16384
_NW = 32
_BPW = _B // _NW
_CHUNK = 128
_NCH = _BPW // _CHUNK
_NBUF = 4


def _sc_encode(idx_all, tab_flat):
    mesh = plsc.VectorSubcoreMesh(core_axis_name="c", subcore_axis_name="s")

    @functools.partial(
        pl.kernel,
        mesh=mesh,
        out_type=jax.ShapeDtypeStruct((_B, _F * _D), jnp.float32),
        scratch_types=[
            pltpu.VMEM((_F * _NCH, _CHUNK), jnp.int32),
            pltpu.VMEM((_NBUF, _BPW, _D), jnp.float32),
            pltpu.SemaphoreType.DMA,
            pltpu.SemaphoreType.DMA,
            pltpu.SemaphoreType.DMA,
            pltpu.SemaphoreType.DMA,
            pltpu.SemaphoreType.DMA,
            pltpu.SemaphoreType.DMA,
            pltpu.SemaphoreType.DMA,
            pltpu.SemaphoreType.DMA,
        ],
        compiler_params=pltpu.CompilerParams(use_tc_tiling_on_sc=False),
    )
    def k(idx_hbm, tab_hbm, out_hbm, idx_v, bufs, *sems):
        gsem = sems[:_NBUF]
        ssem = sems[_NBUF:]
        wid = lax.axis_index("s") * 2 + lax.axis_index("c")
        base = wid * _BPW

        pltpu.sync_copy(idx_hbm.at[wid], idx_v)

        def fire_gathers(f):
            b = f % _NBUF
            return [
                pltpu.async_copy(
                    tab_hbm.at[idx_v.at[f * _NCH + c]],
                    bufs.at[b, pl.ds(c * _CHUNK, _CHUNK)],
                    gsem[b],
                )
                for c in range(_NCH)
            ]

        gh = {}
        sh = {}
        for f in range(_NBUF):
            gh[f] = fire_gathers(f)
        for f in range(_F):
            b = f % _NBUF
            for h in gh.pop(f):
                h.wait()
            sh[f] = pltpu.async_copy(
                bufs.at[b], out_hbm.at[pl.ds(base, _BPW), pl.ds(f * _D, _D)], ssem[b]
            )
            nf = f + _NBUF
            if nf < _F:
                sh.pop(f).wait()
                gh[nf] = fire_gathers(nf)
        for f in range(_F - _NBUF, _F):
            sh.pop(f).wait()

    return k(idx_all, tab_flat)


def kernel(sparse_tensors, tables):
    idx = sparse_tensors.astype(jnp.int32)
    offs = (jnp.arange(_F, dtype=jnp.int32) * _V)[:, None]
    gidx = (
        (idx + offs)
        .reshape(_F, _NW, _NCH, _CHUNK)
        .transpose(1, 0, 2, 3)
        .reshape(_NW, _F * _NCH, _CHUNK)
    )
    tab_flat = tables.reshape(_F * _V, _D)
    return _sc_encode(gidx, tab_flat)
